# baseline (speedup 1.0000x reference)
import functools

import jax
import jax.numpy as jnp
import numpy as np
from jax import lax
from jax.experimental import pallas as pl
from jax.experimental.pallas import tpu as pltpu
from jax.experimental.pallas import tpu_sc as plsc

_NEGATIVE_RATIO = 3.0
_EPS = 1e-06

_NC = 2
_NS = 16
_NW = _NC * _NS
_LANES = 16
_COLS = 512
_CHUNK_ROWS = 8
_TABLE_N = 4096


def _make_loss_table():
    idx = (np.arange(_TABLE_N + 1, dtype=np.uint64) << 18).astype(np.uint32)
    x = idx.view(np.float32).astype(np.float64)
    x0, x1 = x[:-1], x[1:]
    with np.errstate(divide='ignore', invalid='ignore'):
        ent = 1.0 - (x1 * np.log(x1)
                     - np.where(x0 > 0, x0 * np.log(x0), 0.0)) / (x1 - x0)
    ent[~np.isfinite(ent)] = 0.0
    return ent.astype(np.float32)


_LOSS_TABLE = _make_loss_table()


def _worker_id():
    return lax.axis_index("s") * _NC + lax.axis_index("c")


def _loss_vec(tab_v, p, g):
    x = jnp.where(g > 0.5, p, 1.0 - p)
    idx = lax.bitcast_convert_type(x, jnp.int32) >> 18
    return plsc.load_gather(tab_v, [idx])


def _mesh():
    return plsc.VectorSubcoreMesh(
        core_axis_name="c", subcore_axis_name="s",
        num_cores=_NC, num_subcores=_NS)


def _double_buffered(pred_hbm, gt_hbm, mask_hbm, bufs, sems, rows_per_w,
                     compute_chunk, init_carry, after_start=None):
    wid = _worker_id()
    base0 = wid * rows_per_w
    nchunk = rows_per_w // _CHUNK_ROWS

    def start(c, b):
        base = base0 + c * _CHUNK_ROWS
        pltpu.async_copy(pred_hbm.at[pl.ds(base, _CHUNK_ROWS)],
                         bufs[b][0], sems[b])
        pltpu.async_copy(gt_hbm.at[pl.ds(base, _CHUNK_ROWS)],
                         bufs[b][1], sems[b])
        pltpu.async_copy(mask_hbm.at[pl.ds(base, _CHUNK_ROWS)],
                         bufs[b][2], sems[b])

    def wait(b):
        for r in bufs[b]:
            pltpu.make_async_copy(
                pred_hbm.at[pl.ds(0, _CHUNK_ROWS)], r, sems[b]).wait()

    start(0, 0)
    start(1, 1)
    if after_start is not None:
        after_start()

    def pair_body(i, carry):
        wait(0)
        carry = compute_chunk(bufs[0], carry)
        start(2 * i + 2, 0)
        wait(1)
        carry = compute_chunk(bufs[1], carry)
        start(2 * i + 3, 1)
        return carry

    carry = lax.fori_loop(0, nchunk // 2 - 1, pair_body, init_carry)
    wait(0)
    carry = compute_chunk(bufs[0], carry)
    wait(1)
    carry = compute_chunk(bufs[1], carry)
    return carry


_SCRATCH = [
    pltpu.VMEM((_CHUNK_ROWS, _COLS), jnp.float32),
    pltpu.VMEM((_CHUNK_ROWS, _COLS), jnp.float32),
    pltpu.VMEM((_CHUNK_ROWS, _COLS), jnp.float32),
    pltpu.VMEM((_CHUNK_ROWS, _COLS), jnp.float32),
    pltpu.VMEM((_CHUNK_ROWS, _COLS), jnp.float32),
    pltpu.VMEM((_CHUNK_ROWS, _COLS), jnp.float32),
    pltpu.VMEM((_TABLE_N,), jnp.float32),
    pltpu.VMEM((64,), jnp.float32),
    pltpu.SemaphoreType.DMA,
    pltpu.SemaphoreType.DMA,
    pltpu.SemaphoreType.DMA,
]


def _make_main(nrows):
    rows_per_w = nrows // _NW

    @functools.partial(
        pl.kernel, mesh=_mesh(),
        out_type=jax.ShapeDtypeStruct((_NW, 64), jnp.float32),
        scratch_types=_SCRATCH,
        compiler_params=pltpu.CompilerParams(needs_layout_passes=False),
    )
    def main_k(pred_hbm, gt_hbm, mask_hbm, tab_hbm, out_hbm,
               p0, g0, m0, p1, g1, m1, tab_v, acc_v, sem0, sem1, sem_t):
        pltpu.async_copy(tab_hbm, tab_v, sem_t)

        def _wait_table():
            pltpu.make_async_copy(tab_hbm, tab_v, sem_t).wait()

        def compute_chunk(b, carry):
            pv, gv, mv = b

            def row_body(r, inner):
                def col_body(c, inner2):
                    cm, cp, sm, sp = inner2
                    base = pl.multiple_of(c * (8 * _LANES), 8 * _LANES)
                    for u in range(8):
                        off = base + u * _LANES
                        p = pv[r, pl.ds(off, _LANES)]
                        g = gv[r, pl.ds(off, _LANES)]
                        m = mv[r, pl.ds(off, _LANES)]
                        loss = _loss_vec(tab_v, p, g)
                        gm = g * m
                        lm = loss * m
                        glm = g * lm
                        cm = cm + m
                        cp = cp + gm
                        sm = sm + lm
                        sp = sp + glm
                    return (cm, cp, sm, sp)

                return lax.fori_loop(0, _COLS // (8 * _LANES),
                                     col_body, inner)

            return lax.fori_loop(0, _CHUNK_ROWS, row_body, carry)

        zero = jnp.zeros((_LANES,), jnp.float32)
        cm, cp, sm, sp = _double_buffered(
            pred_hbm, gt_hbm, mask_hbm,
            ((p0, g0, m0), (p1, g1, m1)), (sem0, sem1), rows_per_w,
            compute_chunk, (zero, zero, zero, zero),
            after_start=_wait_table)
        acc_v[pl.ds(0, 16)] = cm
        acc_v[pl.ds(16, 16)] = cp
        acc_v[pl.ds(32, 16)] = sm
        acc_v[pl.ds(48, 16)] = sp
        pltpu.sync_copy(acc_v, out_hbm.at[_worker_id()])

    return main_k


def _make_pass2(nrows):
    rows_per_w = nrows // _NW

    @functools.partial(
        pl.kernel, mesh=_mesh(),
        out_type=jax.ShapeDtypeStruct((_NW, 64), jnp.float32),
        scratch_types=_SCRATCH + [pltpu.VMEM((16,), jnp.float32)],
        compiler_params=pltpu.CompilerParams(needs_layout_passes=False),
    )
    def pass2_k(pred_hbm, gt_hbm, mask_hbm, tab_hbm, out_hbm,
                p0, g0, m0, p1, g1, m1, tab_v, acc_v, sem0, sem1, sem_t,
                t_v):
        pltpu.sync_copy(tab_hbm.at[pl.ds(0, _TABLE_N)], tab_v)
        pltpu.sync_copy(tab_hbm.at[pl.ds(_TABLE_N, 16)], t_v)
        t = lax.bitcast_convert_type(t_v[pl.ds(0, 16)], jnp.int32)

        def compute_chunk(b, carry):
            pv, gv, mv = b

            def row_body(r, inner):
                def col_body(c, inner2):
                    cge, cgt, sgt = inner2
                    base = pl.multiple_of(c * (8 * _LANES), 8 * _LANES)
                    for u in range(8):
                        off = base + u * _LANES
                        p = pv[r, pl.ds(off, _LANES)]
                        g = gv[r, pl.ds(off, _LANES)]
                        m = mv[r, pl.ds(off, _LANES)]
                        loss = _loss_vec(tab_v, p, g)
                        v = (m - g * m) * loss
                        vb = lax.bitcast_convert_type(v, jnp.int32)
                        one = jnp.float32(1.0)
                        zero = jnp.float32(0.0)
                        cge = cge + jnp.where(vb >= t, one, zero)
                        cgt = cgt + jnp.where(vb > t, one, zero)
                        sgt = sgt + jnp.where(vb > t, v, zero)
                    return (cge, cgt, sgt)

                return lax.fori_loop(0, _COLS // (8 * _LANES),
                                     col_body, inner)

            return lax.fori_loop(0, _CHUNK_ROWS, row_body, carry)

        zero = jnp.zeros((_LANES,), jnp.float32)
        cge, cgt, sgt = _double_buffered(
            pred_hbm, gt_hbm, mask_hbm,
            ((p0, g0, m0), (p1, g1, m1)), (sem0, sem1), rows_per_w,
            compute_chunk, (zero, zero, zero))
        acc_v[pl.ds(0, 16)] = cge
        acc_v[pl.ds(16, 16)] = cgt
        acc_v[pl.ds(32, 16)] = sgt
        acc_v[pl.ds(48, 16)] = zero
        pltpu.sync_copy(acc_v, out_hbm.at[_worker_id()])

    return pass2_k


_SC_ROWS = 3584
_TC_BLOCK_ROWS = 768


def _make_tc(nrows, start_row):
    nblk = (nrows - start_row) // _TC_BLOCK_ROWS

    def tc_body(p_ref, g_ref, m_ref, out_ref):
        @pl.when(pl.program_id(0) == 0)
        def _init():
            out_ref[...] = jnp.zeros_like(out_ref)

        p = p_ref[...]
        g = g_ref[...]
        m = m_ref[...]
        x = jnp.where(g > 0.5, p, 1.0 - p)
        loss = -jnp.log(x)
        gm = g * m
        lm = loss * m
        glm = g * lm
        for i, q in enumerate((m, gm, lm, glm)):
            out_ref[i] += q.reshape(-1, 8, 128).sum(axis=0)

    off = start_row // _TC_BLOCK_ROWS
    return pl.pallas_call(
        tc_body,
        grid=(nblk,),
        in_specs=[pl.BlockSpec((_TC_BLOCK_ROWS, _COLS),
                               lambda i: (i + off, 0))] * 3,
        out_specs=pl.BlockSpec((4, 8, 128), lambda i: (0, 0, 0)),
        out_shape=jax.ShapeDtypeStruct((4, 8, 128), jnp.float32),
    )


def kernel(pred, gt, mask):
    n = pred.size
    nrows = n // _COLS
    pf = pred.reshape(nrows, _COLS)
    gf = gt.reshape(nrows, _COLS)
    mf = mask.reshape(nrows, _COLS)
    tab = jnp.asarray(_LOSS_TABLE)

    partials = _make_main(_SC_ROWS)(pf, gf, mf, tab)
    tc_sums = _make_tc(nrows, _SC_ROWS)(pf, gf, mf).sum(axis=(1, 2))
    sums = partials.reshape(_NW, 4, _LANES).sum(axis=(0, 2)) + tc_sums
    cnt_m, pos_cnt, sum_ml, pos_sum = sums[0], sums[1], sums[2], sums[3]
    neg_cnt = cnt_m - pos_cnt
    neg_sum = sum_ml - pos_sum
    k = jnp.minimum(neg_cnt, jnp.floor(pos_cnt * _NEGATIVE_RATIO))

    pass2 = _make_pass2(nrows)

    def _all_kept(_):
        return neg_sum

    def _bisect(_):
        def run(t_bits):
            thr = jnp.full((16,), t_bits, jnp.int32)
            tab2 = jnp.concatenate(
                [tab, lax.bitcast_convert_type(thr, jnp.float32)])
            o = pass2(pf, gf, mf, tab2)
            o = o.reshape(_NW, 4, _LANES).sum(axis=(0, 2))
            return o[0], o[1], o[2]

        def cond_fn(c):
            lo, hi = c
            return hi - lo > 1

        def body_fn(c):
            lo, hi = c
            mid = lo + (hi - lo) // 2
            cge, _, _ = run(mid)
            return lax.cond(cge >= k,
                            lambda: (mid, hi),
                            lambda: (lo, mid))

        lo, hi = lax.while_loop(
            cond_fn, body_fn,
            (jnp.int32(0), jnp.int32(0x41700000)))
        _, cnt_gt, sum_gt = run(lo)
        t = lax.bitcast_convert_type(lo, jnp.float32)
        return sum_gt + (k - cnt_gt) * t

    topk_sum = lax.cond(k >= neg_cnt, _all_kept, _bisect, operand=None)
    return (pos_sum + topk_sum) / (pos_cnt + k + _EPS)


# speedup vs baseline: 1.0924x; 1.0924x over previous
import functools

import jax
import jax.numpy as jnp
import numpy as np
from jax import lax
from jax.experimental import pallas as pl
from jax.experimental.pallas import tpu as pltpu
from jax.experimental.pallas import tpu_sc as plsc

_NEGATIVE_RATIO = 3.0
_EPS = 1e-06

_NC = 2
_NS = 16
_NW = _NC * _NS
_LANES = 16
_COLS = 512
_CHUNK_ROWS = 8
_TABLE_N = 4096


def _make_loss_table():
    idx = (np.arange(_TABLE_N + 1, dtype=np.uint64) << 18).astype(np.uint32)
    x = idx.view(np.float32).astype(np.float64)
    x0, x1 = x[:-1], x[1:]
    with np.errstate(divide='ignore', invalid='ignore'):
        ent = 1.0 - (x1 * np.log(x1)
                     - np.where(x0 > 0, x0 * np.log(x0), 0.0)) / (x1 - x0)
    ent[~np.isfinite(ent)] = 0.0
    return ent.astype(np.float32)


_LOSS_TABLE = _make_loss_table()


def _worker_id():
    return lax.axis_index("s") * _NC + lax.axis_index("c")


def _loss_vec(tab_v, p, g):
    x = jnp.where(g > 0.5, p, 1.0 - p)
    idx = lax.bitcast_convert_type(x, jnp.int32) >> 18
    return plsc.load_gather(tab_v, [idx])


def _mesh():
    return plsc.VectorSubcoreMesh(
        core_axis_name="c", subcore_axis_name="s",
        num_cores=_NC, num_subcores=_NS)


def _double_buffered(pred_hbm, gt_hbm, mask_hbm, bufs, sems, rows_per_w,
                     compute_chunk, init_carry, after_start=None):
    wid = _worker_id()
    base0 = wid * rows_per_w
    nchunk = rows_per_w // _CHUNK_ROWS

    def start(c, b):
        base = base0 + c * _CHUNK_ROWS
        pltpu.async_copy(pred_hbm.at[pl.ds(base, _CHUNK_ROWS)],
                         bufs[b][0], sems[b])
        pltpu.async_copy(gt_hbm.at[pl.ds(base, _CHUNK_ROWS)],
                         bufs[b][1], sems[b])
        pltpu.async_copy(mask_hbm.at[pl.ds(base, _CHUNK_ROWS)],
                         bufs[b][2], sems[b])

    def wait(b):
        for r in bufs[b]:
            pltpu.make_async_copy(
                pred_hbm.at[pl.ds(0, _CHUNK_ROWS)], r, sems[b]).wait()

    start(0, 0)
    start(1, 1)
    if after_start is not None:
        after_start()

    def pair_body(i, carry):
        wait(0)
        carry = compute_chunk(bufs[0], carry)
        start(2 * i + 2, 0)
        wait(1)
        carry = compute_chunk(bufs[1], carry)
        start(2 * i + 3, 1)
        return carry

    carry = lax.fori_loop(0, nchunk // 2 - 1, pair_body, init_carry)
    wait(0)
    carry = compute_chunk(bufs[0], carry)
    wait(1)
    carry = compute_chunk(bufs[1], carry)
    return carry


_SCRATCH = [
    pltpu.VMEM((_CHUNK_ROWS, _COLS), jnp.float32),
    pltpu.VMEM((_CHUNK_ROWS, _COLS), jnp.float32),
    pltpu.VMEM((_CHUNK_ROWS, _COLS), jnp.float32),
    pltpu.VMEM((_CHUNK_ROWS, _COLS), jnp.float32),
    pltpu.VMEM((_CHUNK_ROWS, _COLS), jnp.float32),
    pltpu.VMEM((_CHUNK_ROWS, _COLS), jnp.float32),
    pltpu.VMEM((_TABLE_N,), jnp.float32),
    pltpu.VMEM((64,), jnp.float32),
    pltpu.SemaphoreType.DMA,
    pltpu.SemaphoreType.DMA,
    pltpu.SemaphoreType.DMA,
]


def _make_main(nrows):
    rows_per_w = nrows // _NW

    @functools.partial(
        pl.kernel, mesh=_mesh(),
        out_type=jax.ShapeDtypeStruct((_NW, 64), jnp.float32),
        scratch_types=_SCRATCH,
        compiler_params=pltpu.CompilerParams(needs_layout_passes=False),
    )
    def main_k(pred_hbm, gt_hbm, mask_hbm, tab_hbm, out_hbm,
               p0, g0, m0, p1, g1, m1, tab_v, acc_v, sem0, sem1, sem_t):
        pltpu.async_copy(tab_hbm, tab_v, sem_t)

        def _wait_table():
            pltpu.make_async_copy(tab_hbm, tab_v, sem_t).wait()

        def compute_chunk(b, carry):
            pv, gv, mv = b

            def row_body(r, inner):
                def col_body(c, inner2):
                    cm, cp, sm, sp = inner2
                    base = pl.multiple_of(c * (8 * _LANES), 8 * _LANES)
                    for u in range(8):
                        off = base + u * _LANES
                        p = pv[r, pl.ds(off, _LANES)]
                        g = gv[r, pl.ds(off, _LANES)]
                        m = mv[r, pl.ds(off, _LANES)]
                        loss = _loss_vec(tab_v, p, g)
                        gm = g * m
                        lm = loss * m
                        glm = g * lm
                        cm = cm + m
                        cp = cp + gm
                        sm = sm + lm
                        sp = sp + glm
                    return (cm, cp, sm, sp)

                return lax.fori_loop(0, _COLS // (8 * _LANES),
                                     col_body, inner)

            return lax.fori_loop(0, _CHUNK_ROWS, row_body, carry)

        zero = jnp.zeros((_LANES,), jnp.float32)
        cm, cp, sm, sp = _double_buffered(
            pred_hbm, gt_hbm, mask_hbm,
            ((p0, g0, m0), (p1, g1, m1)), (sem0, sem1), rows_per_w,
            compute_chunk, (zero, zero, zero, zero),
            after_start=_wait_table)
        acc_v[pl.ds(0, 16)] = cm
        acc_v[pl.ds(16, 16)] = cp
        acc_v[pl.ds(32, 16)] = sm
        acc_v[pl.ds(48, 16)] = sp
        pltpu.sync_copy(acc_v, out_hbm.at[_worker_id()])

    return main_k


def _make_pass2(nrows):
    rows_per_w = nrows // _NW

    @functools.partial(
        pl.kernel, mesh=_mesh(),
        out_type=jax.ShapeDtypeStruct((_NW, 64), jnp.float32),
        scratch_types=_SCRATCH + [pltpu.VMEM((16,), jnp.float32)],
        compiler_params=pltpu.CompilerParams(needs_layout_passes=False),
    )
    def pass2_k(pred_hbm, gt_hbm, mask_hbm, tab_hbm, out_hbm,
                p0, g0, m0, p1, g1, m1, tab_v, acc_v, sem0, sem1, sem_t,
                t_v):
        pltpu.sync_copy(tab_hbm.at[pl.ds(0, _TABLE_N)], tab_v)
        pltpu.sync_copy(tab_hbm.at[pl.ds(_TABLE_N, 16)], t_v)
        t = lax.bitcast_convert_type(t_v[pl.ds(0, 16)], jnp.int32)

        def compute_chunk(b, carry):
            pv, gv, mv = b

            def row_body(r, inner):
                def col_body(c, inner2):
                    cge, cgt, sgt = inner2
                    base = pl.multiple_of(c * (8 * _LANES), 8 * _LANES)
                    for u in range(8):
                        off = base + u * _LANES
                        p = pv[r, pl.ds(off, _LANES)]
                        g = gv[r, pl.ds(off, _LANES)]
                        m = mv[r, pl.ds(off, _LANES)]
                        loss = _loss_vec(tab_v, p, g)
                        v = (m - g * m) * loss
                        vb = lax.bitcast_convert_type(v, jnp.int32)
                        one = jnp.float32(1.0)
                        zero = jnp.float32(0.0)
                        cge = cge + jnp.where(vb >= t, one, zero)
                        cgt = cgt + jnp.where(vb > t, one, zero)
                        sgt = sgt + jnp.where(vb > t, v, zero)
                    return (cge, cgt, sgt)

                return lax.fori_loop(0, _COLS // (8 * _LANES),
                                     col_body, inner)

            return lax.fori_loop(0, _CHUNK_ROWS, row_body, carry)

        zero = jnp.zeros((_LANES,), jnp.float32)
        cge, cgt, sgt = _double_buffered(
            pred_hbm, gt_hbm, mask_hbm,
            ((p0, g0, m0), (p1, g1, m1)), (sem0, sem1), rows_per_w,
            compute_chunk, (zero, zero, zero))
        acc_v[pl.ds(0, 16)] = cge
        acc_v[pl.ds(16, 16)] = cgt
        acc_v[pl.ds(32, 16)] = sgt
        acc_v[pl.ds(48, 16)] = zero
        pltpu.sync_copy(acc_v, out_hbm.at[_worker_id()])

    return pass2_k


_SC_ROWS = 3584
_TC_BLOCK_ROWS = 512


def _make_tc(nrows, start_row):
    nblk = (nrows - start_row) // _TC_BLOCK_ROWS

    def tc_body(p_ref, g_ref, m_ref, out_ref):
        @pl.when(pl.program_id(0) == 0)
        def _init():
            out_ref[...] = jnp.zeros_like(out_ref)

        p = p_ref[...]
        g = g_ref[...]
        m = m_ref[...]
        x = jnp.where(g > 0.5, p, 1.0 - p)
        loss = -jnp.log(x)
        gm = g * m
        lm = loss * m
        glm = g * lm
        for i, q in enumerate((m, gm, lm, glm)):
            out_ref[i] += q.reshape(-1, 8, 128).sum(axis=0)

    off = start_row // _TC_BLOCK_ROWS
    return pl.pallas_call(
        tc_body,
        grid=(nblk,),
        in_specs=[pl.BlockSpec((_TC_BLOCK_ROWS, _COLS),
                               lambda i: (i + off, 0))] * 3,
        out_specs=pl.BlockSpec((4, 8, 128), lambda i: (0, 0, 0)),
        out_shape=jax.ShapeDtypeStruct((4, 8, 128), jnp.float32),
    )


def _combine_kernel(sc_ref, tc_ref, out_ref):
    scp = sc_ref[...]
    gi = lax.broadcasted_iota(jnp.int32, (32, 64), 1) >> 4
    tcs = jnp.sum(tc_ref[...], axis=(1, 2))

    def part(j):
        return jnp.sum(jnp.where(gi == j, scp, 0.0)) + tcs[j]

    cnt_m, pos_cnt, sum_ml, pos_sum = part(0), part(1), part(2), part(3)
    neg_cnt = cnt_m - pos_cnt
    neg_sum = sum_ml - pos_sum
    k = jnp.minimum(neg_cnt, jnp.floor(pos_cnt * _NEGATIVE_RATIO))
    common = (pos_sum + neg_sum) / (pos_cnt + k + _EPS)
    flag = jnp.where(k >= neg_cnt, 1.0, 0.0)
    ri = lax.broadcasted_iota(jnp.int32, (8, 128), 0)
    ci = lax.broadcasted_iota(jnp.int32, (8, 128), 1)

    def ind(j):
        return ((ri == 0) & (ci == j)).astype(jnp.float32)

    out_ref[...] = (ind(0) * common + ind(1) * flag + ind(2) * k
                    + ind(3) * pos_sum + ind(4) * pos_cnt)


def kernel(pred, gt, mask):
    n = pred.size
    nrows = n // _COLS
    pf = pred.reshape(nrows, _COLS)
    gf = gt.reshape(nrows, _COLS)
    mf = mask.reshape(nrows, _COLS)
    tab = jnp.asarray(_LOSS_TABLE)

    partials = _make_main(_SC_ROWS)(pf, gf, mf, tab)
    tc_part = _make_tc(nrows, _SC_ROWS)(pf, gf, mf)
    comb = pl.pallas_call(
        _combine_kernel,
        out_shape=jax.ShapeDtypeStruct((8, 128), jnp.float32),
    )(partials, tc_part)
    common = comb[0, 0]
    flag = comb[0, 1]
    k = comb[0, 2]
    pos_sum = comb[0, 3]
    pos_cnt = comb[0, 4]

    pass2 = _make_pass2(nrows)

    def _all_kept(_):
        return common

    def _bisect(_):
        def run(t_bits):
            thr = jnp.full((16,), t_bits, jnp.int32)
            tab2 = jnp.concatenate(
                [tab, lax.bitcast_convert_type(thr, jnp.float32)])
            o = pass2(pf, gf, mf, tab2)
            o = o.reshape(_NW, 4, _LANES).sum(axis=(0, 2))
            return o[0], o[1], o[2]

        def cond_fn(c):
            lo, hi = c
            return hi - lo > 1

        def body_fn(c):
            lo, hi = c
            mid = lo + (hi - lo) // 2
            cge, _, _ = run(mid)
            return lax.cond(cge >= k,
                            lambda: (mid, hi),
                            lambda: (lo, mid))

        lo, hi = lax.while_loop(
            cond_fn, body_fn,
            (jnp.int32(0), jnp.int32(0x41700000)))
        _, cnt_gt, sum_gt = run(lo)
        t = lax.bitcast_convert_type(lo, jnp.float32)
        topk_sum = sum_gt + (k - cnt_gt) * t
        return (pos_sum + topk_sum) / (pos_cnt + k + _EPS)

    return lax.cond(flag > 0.5, _all_kept, _bisect, operand=None)
